# baseline (speedup 1.0000x reference)
import functools

import jax
import jax.numpy as jnp
from jax.experimental import pallas as pl
from jax.experimental.pallas import tpu as pltpu

_EPS = 1e-5
_LANE = 128


def _ru(v, m):
    return (v + m - 1) // m * m


def _rowsum8(v):
    r = v.shape[0]
    while r > 8 and r % 2 == 0:
        half = r // 2
        v = v[:half] + v[half:]
        r = half
    if r > 8:
        v = jnp.concatenate(
            [jnp.sum(v, axis=0, keepdims=True),
             jnp.zeros((7, v.shape[1]), v.dtype)], axis=0)
    return v


def _fused_kernel(x_ref, w1_ref, b1_ref, g1_ref, be1_ref, w2_ref, b2_ref,
                  o_ref,
                  xb_ref, gram_ref, cs1_ref, w1s_ref, sh1_ref,
                  zs_ref, zq_ref, w2s_ref, c2_ref,
                  *, bm, t_steps, m, masked):
    p = pl.program_id(0)
    t = pl.program_id(1)
    inv_m = jnp.float32(1.0 / m)
    rows = pl.ds(t * bm, bm)

    @pl.when(p == 0)
    def _phase0():
        @pl.when(t == 0)
        def _():
            gram_ref[...] = jnp.zeros_like(gram_ref)
            cs1_ref[...] = jnp.zeros_like(cs1_ref)

        x = x_ref[...]
        xb = x.astype(jnp.bfloat16)
        xb_ref[rows, :] = xb
        gram_ref[...] += jax.lax.dot_general(
            xb, xb, (((0,), (0,)), ((), ())),
            preferred_element_type=jnp.float32)
        cs1_ref[...] += _rowsum8(x)

    @pl.when(jnp.logical_and(p == 1, t == 0))
    def _stats1():
        w1b = w1_ref[...]
        w1f = w1b.astype(jnp.float32)
        cs = jnp.sum(cs1_ref[...], axis=0, keepdims=True)
        sh0 = jnp.dot(cs.astype(jnp.bfloat16), w1b,
                      preferred_element_type=jnp.float32)
        d = jnp.dot(gram_ref[...].astype(jnp.bfloat16), w1b,
                    preferred_element_type=jnp.float32)
        sq0 = jnp.sum(_rowsum8(w1f * d), axis=0, keepdims=True)
        b1 = b1_ref[...]
        mean1 = sh0 * inv_m + b1
        ex2 = (sq0 + 2.0 * b1 * sh0) * inv_m + b1 * b1
        var1 = jnp.maximum(ex2 - mean1 * mean1, 0.0)
        scale1 = g1_ref[...] * jax.lax.rsqrt(var1 + _EPS)
        sh1_ref[...] = (b1 - mean1) * scale1 + be1_ref[...]
        w1s_ref[...] = (w1f * scale1).astype(jnp.bfloat16)
        zs_ref[...] = jnp.zeros_like(zs_ref)
        zq_ref[...] = jnp.zeros_like(zq_ref)

    @pl.when(p == 1)
    def _phase1():
        xb = xb_ref[rows, :]
        hs = jnp.dot(xb, w1s_ref[...], preferred_element_type=jnp.float32)
        a = jnp.maximum(hs + sh1_ref[...], 0.0)
        z = jnp.dot(a.astype(jnp.bfloat16), w2_ref[...],
                    preferred_element_type=jnp.float32)
        if masked:
            row = t * bm + jax.lax.broadcasted_iota(jnp.int32, (bm, 1), 0)
            z = z * (row < m).astype(jnp.float32)
        zs_ref[...] += _rowsum8(z)
        zq_ref[...] += _rowsum8(z * z)

    @pl.when(jnp.logical_and(p == 2, t == 0))
    def _stats2():
        mz = jnp.sum(zs_ref[...], axis=0, keepdims=True) * inv_m
        vz = jnp.maximum(
            jnp.sum(zq_ref[...], axis=0, keepdims=True) * inv_m - mz * mz, 0.0)
        rstd2 = jax.lax.rsqrt(vz + _EPS)
        w2s_ref[...] = (w2_ref[...].astype(jnp.float32) * rstd2
                        ).astype(jnp.bfloat16)
        c2_ref[...] = -mz * rstd2

    @pl.when(p == 2)
    def _phase2():
        xb = xb_ref[rows, :]
        hs = jnp.dot(xb, w1s_ref[...], preferred_element_type=jnp.float32)
        a = jnp.maximum(hs + sh1_ref[...], 0.0)
        z = jnp.dot(a.astype(jnp.bfloat16), w2s_ref[...],
                    preferred_element_type=jnp.float32)
        o_ref[...] = z + c2_ref[...]


def kernel(x, w1, b1, g1, be1, w2, b2):
    B, N, C = x.shape
    H = w1.shape[1]
    O = w2.shape[1]
    M = B * N

    Cp = _ru(C, _LANE)
    Hp = _ru(H, _LANE)
    Op = _ru(O, _LANE)
    if (Cp, Hp, Op) == (C, H, O):
        w1b = w1.astype(jnp.bfloat16)
        w2b = w2.astype(jnp.bfloat16)
        b1p, g1p, be1p, b2p = b1, g1, be1, b2
    else:
        w1b = jnp.zeros((Cp, Hp), jnp.bfloat16).at[:C, :H].set(w1.astype(jnp.bfloat16))
        b1p = jnp.zeros((1, Hp), jnp.float32).at[:, :H].set(b1)
        g1p = jnp.ones((1, Hp), jnp.float32).at[:, :H].set(g1)
        be1p = jnp.zeros((1, Hp), jnp.float32).at[:, :H].set(be1)
        w2b = jnp.zeros((Hp, Op), jnp.bfloat16).at[:H, :O].set(w2.astype(jnp.bfloat16))
        b2p = jnp.zeros((1, Op), jnp.float32).at[:, :O].set(b2)

    bm = min(4096, max(16, 1 << (M - 1).bit_length()))
    t_steps = -(-M // bm)
    Mp = t_steps * bm
    masked = Mp != M

    x2d = x.reshape(M, C).astype(jnp.float32)
    if Mp != M or Cp != C:
        x2d = jnp.zeros((Mp, Cp), jnp.float32).at[:M, :C].set(x2d)

    def full(a):
        return pl.BlockSpec(a.shape, lambda p, t: (0,) * a.ndim)

    x_spec = pl.BlockSpec((bm, Cp), lambda p, t: (jnp.where(p == 0, t, 0), 0))
    o_spec = pl.BlockSpec((bm, Op), lambda p, t: (jnp.where(p == 2, t, 0), 0))

    out_p = pl.pallas_call(
        functools.partial(_fused_kernel, bm=bm, t_steps=t_steps, m=M,
                          masked=masked),
        out_shape=jax.ShapeDtypeStruct((Mp, Op), jnp.float32),
        grid=(3, t_steps),
        in_specs=[x_spec, full(w1b), full(b1p), full(g1p), full(be1p),
                  full(w2b), full(b2p)],
        out_specs=o_spec,
        scratch_shapes=[
            pltpu.VMEM((Mp, Cp), jnp.bfloat16),
            pltpu.VMEM((Cp, Cp), jnp.float32),
            pltpu.VMEM((8, Cp), jnp.float32),
            pltpu.VMEM((Cp, Hp), jnp.bfloat16),
            pltpu.VMEM((1, Hp), jnp.float32),
            pltpu.VMEM((8, Op), jnp.float32),
            pltpu.VMEM((8, Op), jnp.float32),
            pltpu.VMEM((Hp, Op), jnp.bfloat16),
            pltpu.VMEM((1, Op), jnp.float32),
        ],
        compiler_params=pltpu.CompilerParams(
            dimension_semantics=("arbitrary", "arbitrary"),
            vmem_limit_bytes=48 * 1024 * 1024),
    )(x2d, w1b, b1p, g1p, be1p, w2b, b2p)

    if (Mp, Op) == (M, O):
        return out_p.reshape(B, N, O)
    return out_p[:M, :O].reshape(B, N, O)


# speedup vs baseline: 1.2525x; 1.2525x over previous
import functools

import jax
import jax.numpy as jnp
from jax.experimental import pallas as pl
from jax.experimental.pallas import tpu as pltpu

_EPS = 1e-5
_LANE = 128


def _ru(v, m):
    return (v + m - 1) // m * m


def _rowsum8(v):
    r = v.shape[0]
    while r > 8 and r % 2 == 0:
        half = r // 2
        v = v[:half] + v[half:]
        r = half
    if r > 8:
        v = jnp.concatenate(
            [jnp.sum(v, axis=0, keepdims=True),
             jnp.zeros((7, v.shape[1]), v.dtype)], axis=0)
    return v


def _fused_kernel(x_ref, w1_ref, b1_ref, g1_ref, be1_ref, w2_ref, b2_ref,
                  o_ref,
                  hb_ref, gram_ref, cs1_ref, sc1_ref, sh1_ref,
                  zs_ref, zq_ref, w2s_ref, c2_ref,
                  *, bm, t_steps, m, masked):
    p = pl.program_id(0)
    t = pl.program_id(1)
    inv_m = jnp.float32(1.0 / m)
    rows = pl.ds(t * bm, bm)

    @pl.when(p == 0)
    def _phase0():
        @pl.when(t == 0)
        def _():
            gram_ref[...] = jnp.zeros_like(gram_ref)
            cs1_ref[...] = jnp.zeros_like(cs1_ref)

        x = x_ref[...]
        xb = x.astype(jnp.bfloat16)
        h = jnp.dot(xb, w1_ref[...], preferred_element_type=jnp.float32)
        hb_ref[rows, :] = h.astype(jnp.bfloat16)
        gram_ref[...] += jax.lax.dot_general(
            xb, xb, (((0,), (0,)), ((), ())),
            preferred_element_type=jnp.float32)
        cs1_ref[...] += _rowsum8(x)

    @pl.when(jnp.logical_and(p == 1, t == 0))
    def _stats1():
        w1b = w1_ref[...]
        w1f = w1b.astype(jnp.float32)
        cs = jnp.sum(cs1_ref[...], axis=0, keepdims=True)
        sh0 = jnp.dot(cs.astype(jnp.bfloat16), w1b,
                      preferred_element_type=jnp.float32)
        d = jnp.dot(gram_ref[...].astype(jnp.bfloat16), w1b,
                    preferred_element_type=jnp.float32)
        sq0 = jnp.sum(_rowsum8(w1f * d), axis=0, keepdims=True)
        b1 = b1_ref[...]
        mean1 = sh0 * inv_m + b1
        ex2 = (sq0 + 2.0 * b1 * sh0) * inv_m + b1 * b1
        var1 = jnp.maximum(ex2 - mean1 * mean1, 0.0)
        scale1 = g1_ref[...] * jax.lax.rsqrt(var1 + _EPS)
        sc1_ref[...] = scale1.astype(jnp.bfloat16)
        sh1_ref[...] = ((b1 - mean1) * scale1 + be1_ref[...]
                        ).astype(jnp.bfloat16)
        zs_ref[...] = jnp.zeros_like(zs_ref)
        zq_ref[...] = jnp.zeros_like(zq_ref)

    @pl.when(p == 1)
    def _phase1():
        hb = hb_ref[rows, :]
        a = jnp.maximum(hb * sc1_ref[...] + sh1_ref[...],
                        jnp.bfloat16(0.0))
        z = jnp.dot(a, w2_ref[...], preferred_element_type=jnp.float32)
        if masked:
            row = t * bm + jax.lax.broadcasted_iota(jnp.int32, (bm, 1), 0)
            z = z * (row < m).astype(jnp.float32)
        zs_ref[...] += _rowsum8(z)
        zq_ref[...] += _rowsum8(z * z)

    @pl.when(jnp.logical_and(p == 2, t == 0))
    def _stats2():
        mz = jnp.sum(zs_ref[...], axis=0, keepdims=True) * inv_m
        vz = jnp.maximum(
            jnp.sum(zq_ref[...], axis=0, keepdims=True) * inv_m - mz * mz, 0.0)
        rstd2 = jax.lax.rsqrt(vz + _EPS)
        w2s_ref[...] = (w2_ref[...].astype(jnp.float32) * rstd2
                        ).astype(jnp.bfloat16)
        c2_ref[...] = -mz * rstd2

    @pl.when(p == 2)
    def _phase2():
        hb = hb_ref[rows, :]
        a = jnp.maximum(hb * sc1_ref[...] + sh1_ref[...],
                        jnp.bfloat16(0.0))
        z = jnp.dot(a, w2s_ref[...], preferred_element_type=jnp.float32)
        o_ref[...] = z + c2_ref[...]


def kernel(x, w1, b1, g1, be1, w2, b2):
    B, N, C = x.shape
    H = w1.shape[1]
    O = w2.shape[1]
    M = B * N

    Cp = _ru(C, _LANE)
    Hp = _ru(H, _LANE)
    Op = _ru(O, _LANE)
    if (Cp, Hp, Op) == (C, H, O):
        w1b = w1.astype(jnp.bfloat16)
        w2b = w2.astype(jnp.bfloat16)
        b1p, g1p, be1p, b2p = b1, g1, be1, b2
    else:
        w1b = jnp.zeros((Cp, Hp), jnp.bfloat16).at[:C, :H].set(w1.astype(jnp.bfloat16))
        b1p = jnp.zeros((1, Hp), jnp.float32).at[:, :H].set(b1)
        g1p = jnp.ones((1, Hp), jnp.float32).at[:, :H].set(g1)
        be1p = jnp.zeros((1, Hp), jnp.float32).at[:, :H].set(be1)
        w2b = jnp.zeros((Hp, Op), jnp.bfloat16).at[:H, :O].set(w2.astype(jnp.bfloat16))
        b2p = jnp.zeros((1, Op), jnp.float32).at[:, :O].set(b2)

    bm = min(4096, max(16, 1 << (M - 1).bit_length()))
    t_steps = -(-M // bm)
    Mp = t_steps * bm
    masked = Mp != M

    x2d = x.reshape(M, C).astype(jnp.float32)
    if Mp != M or Cp != C:
        x2d = jnp.zeros((Mp, Cp), jnp.float32).at[:M, :C].set(x2d)

    def full(a):
        return pl.BlockSpec(a.shape, lambda p, t: (0,) * a.ndim)

    x_spec = pl.BlockSpec((bm, Cp), lambda p, t: (jnp.where(p == 0, t, 0), 0))
    o_spec = pl.BlockSpec((bm, Op), lambda p, t: (jnp.where(p == 2, t, 0), 0))

    out_p = pl.pallas_call(
        functools.partial(_fused_kernel, bm=bm, t_steps=t_steps, m=M,
                          masked=masked),
        out_shape=jax.ShapeDtypeStruct((Mp, Op), jnp.float32),
        grid=(3, t_steps),
        in_specs=[x_spec, full(w1b), full(b1p), full(g1p), full(be1p),
                  full(w2b), full(b2p)],
        out_specs=o_spec,
        scratch_shapes=[
            pltpu.VMEM((Mp, Hp), jnp.bfloat16),
            pltpu.VMEM((Cp, Cp), jnp.float32),
            pltpu.VMEM((8, Cp), jnp.float32),
            pltpu.VMEM((1, Hp), jnp.bfloat16),
            pltpu.VMEM((1, Hp), jnp.bfloat16),
            pltpu.VMEM((8, Op), jnp.float32),
            pltpu.VMEM((8, Op), jnp.float32),
            pltpu.VMEM((Hp, Op), jnp.bfloat16),
            pltpu.VMEM((1, Op), jnp.float32),
        ],
        compiler_params=pltpu.CompilerParams(
            dimension_semantics=("arbitrary", "arbitrary"),
            vmem_limit_bytes=48 * 1024 * 1024),
    )(x2d, w1b, b1p, g1p, be1p, w2b, b2p)

    if (Mp, Op) == (M, O):
        return out_p.reshape(B, N, O)
    return out_p[:M, :O].reshape(B, N, O)


# speedup vs baseline: 1.5073x; 1.2035x over previous
import functools

import jax
import jax.numpy as jnp
from jax.experimental import pallas as pl
from jax.experimental.pallas import tpu as pltpu

_EPS = 1e-5
_LANE = 128


def _ru(v, m):
    return (v + m - 1) // m * m


def _rowsum8(v):
    r = v.shape[0]
    while r > 8 and r % 2 == 0:
        half = r // 2
        v = v[:half] + v[half:]
        r = half
    if r > 8:
        v = jnp.concatenate(
            [jnp.sum(v, axis=0, keepdims=True),
             jnp.zeros((7, v.shape[1]), v.dtype)], axis=0)
    return v


def _fused_kernel(x_ref, w1_ref, b1_ref, g1_ref, be1_ref, w2_ref, b2_ref,
                  o_ref,
                  hb_ref, gram_ref, cs1_ref, sc1_ref, sh1_ref,
                  zs_ref, zq_ref, w2s_ref, c2_ref,
                  *, bm, t_steps, m, masked):
    p = pl.program_id(0)
    t = pl.program_id(1)
    inv_m = jnp.float32(1.0 / m)
    rows = pl.ds(t * bm, bm)

    @pl.when(p == 0)
    def _phase0():
        @pl.when(t == 0)
        def _():
            gram_ref[...] = jnp.zeros_like(gram_ref)
            cs1_ref[...] = jnp.zeros_like(cs1_ref)

        x = x_ref[...]
        xb = x.astype(jnp.bfloat16)
        h = jnp.dot(xb, w1_ref[...], preferred_element_type=jnp.float32)
        hb_ref[rows, :] = h.astype(jnp.bfloat16)
        gram_ref[...] += jax.lax.dot_general(
            xb, xb, (((0,), (0,)), ((), ())),
            preferred_element_type=jnp.float32)
        cs1_ref[...] += _rowsum8(x)

    @pl.when(jnp.logical_and(p == 1, t == 0))
    def _stats1():
        w1b = w1_ref[...]
        w1f = w1b.astype(jnp.float32)
        cs = jnp.sum(cs1_ref[...], axis=0, keepdims=True)
        sh0 = jnp.dot(cs.astype(jnp.bfloat16), w1b,
                      preferred_element_type=jnp.float32)
        d = jnp.dot(gram_ref[...].astype(jnp.bfloat16), w1b,
                    preferred_element_type=jnp.float32)
        sq0 = jnp.sum(_rowsum8(w1f * d), axis=0, keepdims=True)
        b1 = b1_ref[...]
        mean1 = sh0 * inv_m + b1
        ex2 = (sq0 + 2.0 * b1 * sh0) * inv_m + b1 * b1
        var1 = jnp.maximum(ex2 - mean1 * mean1, 0.0)
        scale1 = g1_ref[...] * jax.lax.rsqrt(var1 + _EPS)
        sc1_ref[...] = scale1.astype(jnp.bfloat16)
        sh1_ref[...] = ((b1 - mean1) * scale1 + be1_ref[...]
                        ).astype(jnp.bfloat16)
        zs_ref[...] = jnp.zeros_like(zs_ref)
        zq_ref[...] = jnp.zeros_like(zq_ref)

    @pl.when(p == 1)
    def _phase1():
        hb = hb_ref[rows, :]
        a = jnp.maximum(hb * sc1_ref[...] + sh1_ref[...],
                        jnp.bfloat16(0.0))
        z = jnp.dot(a, w2_ref[...], preferred_element_type=jnp.float32)
        if masked:
            row = t * bm + jax.lax.broadcasted_iota(jnp.int32, (bm, 1), 0)
            z = z * (row < m).astype(jnp.float32)
        zs_ref[...] += _rowsum8(z)
        zq_ref[...] += _rowsum8(z * z)

    @pl.when(jnp.logical_and(p == 2, t == 0))
    def _stats2():
        mz = jnp.sum(zs_ref[...], axis=0, keepdims=True) * inv_m
        vz = jnp.maximum(
            jnp.sum(zq_ref[...], axis=0, keepdims=True) * inv_m - mz * mz, 0.0)
        rstd2 = jax.lax.rsqrt(vz + _EPS)
        w2s_ref[...] = (w2_ref[...].astype(jnp.float32) * rstd2
                        ).astype(jnp.bfloat16)
        c2_ref[...] = -mz * rstd2

    @pl.when(p == 2)
    def _phase2():
        hb = hb_ref[rows, :]
        a = jnp.maximum(hb * sc1_ref[...] + sh1_ref[...],
                        jnp.bfloat16(0.0))
        z = jnp.dot(a, w2s_ref[...], preferred_element_type=jnp.float32)
        o_ref[...] = z + c2_ref[...]


def kernel(x, w1, b1, g1, be1, w2, b2):
    B, N, C = x.shape
    H = w1.shape[1]
    O = w2.shape[1]
    M = B * N

    Cp = _ru(C, _LANE)
    Hp = _ru(H, _LANE)
    Op = _ru(O, _LANE)
    if (Cp, Hp, Op) == (C, H, O):
        w1b = w1.astype(jnp.bfloat16)
        w2b = w2.astype(jnp.bfloat16)
        b1p, g1p, be1p, b2p = b1, g1, be1, b2
    else:
        w1b = jnp.zeros((Cp, Hp), jnp.bfloat16).at[:C, :H].set(w1.astype(jnp.bfloat16))
        b1p = jnp.zeros((1, Hp), jnp.float32).at[:, :H].set(b1)
        g1p = jnp.ones((1, Hp), jnp.float32).at[:, :H].set(g1)
        be1p = jnp.zeros((1, Hp), jnp.float32).at[:, :H].set(be1)
        w2b = jnp.zeros((Hp, Op), jnp.bfloat16).at[:H, :O].set(w2.astype(jnp.bfloat16))
        b2p = jnp.zeros((1, Op), jnp.float32).at[:, :O].set(b2)

    bm = min(8192, max(16, 1 << (M - 1).bit_length()))
    t_steps = -(-M // bm)
    Mp = t_steps * bm
    masked = Mp != M

    x2d = x.reshape(M, C).astype(jnp.float32)
    if Mp != M or Cp != C:
        x2d = jnp.zeros((Mp, Cp), jnp.float32).at[:M, :C].set(x2d)

    def full(a):
        return pl.BlockSpec(a.shape, lambda p, t: (0,) * a.ndim)

    x_spec = pl.BlockSpec((bm, Cp), lambda p, t: (jnp.where(p == 0, t, 0), 0))
    o_spec = pl.BlockSpec((bm, Op), lambda p, t: (jnp.where(p == 2, t, 0), 0))

    out_p = pl.pallas_call(
        functools.partial(_fused_kernel, bm=bm, t_steps=t_steps, m=M,
                          masked=masked),
        out_shape=jax.ShapeDtypeStruct((Mp, Op), jnp.float32),
        grid=(3, t_steps),
        in_specs=[x_spec, full(w1b), full(b1p), full(g1p), full(be1p),
                  full(w2b), full(b2p)],
        out_specs=o_spec,
        scratch_shapes=[
            pltpu.VMEM((Mp, Hp), jnp.bfloat16),
            pltpu.VMEM((Cp, Cp), jnp.float32),
            pltpu.VMEM((8, Cp), jnp.float32),
            pltpu.VMEM((1, Hp), jnp.bfloat16),
            pltpu.VMEM((1, Hp), jnp.bfloat16),
            pltpu.VMEM((8, Op), jnp.float32),
            pltpu.VMEM((8, Op), jnp.float32),
            pltpu.VMEM((Hp, Op), jnp.bfloat16),
            pltpu.VMEM((1, Op), jnp.float32),
        ],
        compiler_params=pltpu.CompilerParams(
            dimension_semantics=("arbitrary", "arbitrary"),
            vmem_limit_bytes=56 * 1024 * 1024),
    )(x2d, w1b, b1p, g1p, be1p, w2b, b2p)

    if (Mp, Op) == (M, O):
        return out_p.reshape(B, N, O)
    return out_p[:M, :O].reshape(B, N, O)
